# baseline (speedup 1.0000x reference)
import functools

import jax
import jax.numpy as jnp
from jax import lax
from jax.experimental import pallas as pl
from jax.experimental.pallas import tpu as pltpu
from jax.experimental.pallas import tpu_sc as plsc

B = 16384
F = 32
NC = 2
NS = 16
L = 16
NW = NC * NS
BPW = B // NW
CHUNK = 128
NCHUNK = BPW // CHUNK
NGROUP = BPW // L
REG_C = 0.1
INV_LN2 = 1.4426950408889634


@functools.partial(
    pl.kernel,
    mesh=plsc.VectorSubcoreMesh(core_axis_name="c", subcore_axis_name="s"),
    compiler_params=pltpu.CompilerParams(
        needs_layout_passes=False, use_tc_tiling_on_sc=False),
    out_type=[
        jax.ShapeDtypeStruct((B,), jnp.float32),
        jax.ShapeDtypeStruct((B,), jnp.float32),
        jax.ShapeDtypeStruct((NW, L), jnp.float32),
    ],
    scratch_types=[
        pltpu.VMEM((NCHUNK, CHUNK), jnp.int32),
        pltpu.VMEM((NCHUNK, CHUNK), jnp.int32),
        pltpu.VMEM((NCHUNK, CHUNK), jnp.int32),
        pltpu.VMEM((BPW, F), jnp.float32),
        pltpu.VMEM((BPW, F), jnp.float32),
        pltpu.VMEM((BPW, F), jnp.float32),
        pltpu.VMEM((BPW,), jnp.float32),
        pltpu.VMEM((BPW,), jnp.float32),
        pltpu.VMEM((L,), jnp.float32),
        pltpu.SemaphoreType.DMA,
    ],
)
def _sc_bpr(U_hbm, V_hbm, u_hbm, i_hbm, j_hbm,
            yui_hbm, yuj_hbm, regp_hbm,
            idx_u, idx_i, idx_j, rows_u, rows_i, rows_j,
            yui_v, yuj_v, regp_v, sem):
    wid = lax.axis_index("s") * NC + lax.axis_index("c")
    base = wid * BPW
    crow = wid * NCHUNK

    pltpu.sync_copy(u_hbm.at[pl.ds(crow, NCHUNK)], idx_u)
    pltpu.sync_copy(i_hbm.at[pl.ds(crow, NCHUNK)], idx_i)
    pltpu.sync_copy(j_hbm.at[pl.ds(crow, NCHUNK)], idx_j)

    copies = []
    for c in range(NCHUNK):
        sl = pl.ds(c * CHUNK, CHUNK)
        copies.append(pltpu.async_copy(U_hbm.at[idx_u.at[c]], rows_u.at[sl], sem))
        copies.append(pltpu.async_copy(V_hbm.at[idx_i.at[c]], rows_i.at[sl], sem))
        copies.append(pltpu.async_copy(V_hbm.at[idx_j.at[c]], rows_j.at[sl], sem))
    for cp in copies:
        cp.wait()

    lane = lax.iota(jnp.int32, L)
    last = lane == (L - 1)

    def row_body(r, reg_acc):
        u0 = rows_u[r, pl.ds(0, L)]
        u1 = rows_u[r, pl.ds(L, L)]
        vi0 = rows_i[r, pl.ds(0, L)]
        vi1 = rows_i[r, pl.ds(L, L)]
        vj0 = rows_j[r, pl.ds(0, L)]
        vj1 = rows_j[r, pl.ds(L, L)]
        cum_ui = plsc.cumsum(u0 * vi0 + u1 * vi1)
        cum_uj = plsc.cumsum(u0 * vj0 + u1 * vj1)
        ridx = jnp.full((L,), 0, jnp.int32) + r
        plsc.store_scatter(yui_v, [ridx], cum_ui, mask=last)
        plsc.store_scatter(yuj_v, [ridx], cum_uj, mask=last)
        return reg_acc + ((u0 * u0 + u1 * u1)
                          + (vi0 * vi0 + vi1 * vi1)
                          + (vj0 * vj0 + vj1 * vj1))

    reg_acc = lax.fori_loop(0, BPW, row_body, jnp.zeros((L,), jnp.float32))
    regp_v[...] = reg_acc

    pltpu.sync_copy(yui_v, yui_hbm.at[pl.ds(base, BPW)])
    pltpu.sync_copy(yuj_v, yuj_hbm.at[pl.ds(base, BPW)])
    pltpu.sync_copy(regp_v, regp_hbm.at[wid])


def _loss_body(yui_ref, yuj_ref, regp_ref, out_ref):
    d = yui_ref[...] - yuj_ref[...]
    nls = jnp.log(1.0 + jnp.exp(-d)) * INV_LN2
    out_ref[0, 0] = REG_C * jnp.sum(regp_ref[...]) + jnp.sum(nls)


_loss_call = pl.pallas_call(
    _loss_body,
    out_shape=jax.ShapeDtypeStruct((1, 1), jnp.float32),
    out_specs=pl.BlockSpec(memory_space=pltpu.SMEM),
)


def kernel(U, V, u, i, j):
    u2 = u.reshape(B // CHUNK, CHUNK)
    i2 = i.reshape(B // CHUNK, CHUNK)
    j2 = j.reshape(B // CHUNK, CHUNK)
    y_ui, y_uj, regp = _sc_bpr(U, V, u2, i2, j2)
    loss = _loss_call(y_ui.reshape(B // 128, 128), y_uj.reshape(B // 128, 128), regp)
    return y_ui, y_uj, loss.reshape(())
